# Initial kernel scaffold; baseline (speedup 1.0000x reference)
import jax
import jax.numpy as jnp
from jax.experimental import pallas as pl


def kernel(indices, table):
    raise NotImplementedError("write your pallas kernel here")


# speedup vs baseline: 2.9641x; 2.9641x over previous
import functools

import jax
import jax.numpy as jnp
from jax import lax
from jax.experimental import pallas as pl
from jax.experimental.pallas import tpu as pltpu
from jax.experimental.pallas import tpu_sc as plsc

VOCAB = 100000
EMBED = 128
BATCH = 4096
SEQ = 50

NC = 2
NS = 16
NW = NC * NS
B = BATCH * SEQ
BPW = B // NW
C = 128
NCHUNK = BPW // C

_mesh = plsc.VectorSubcoreMesh(core_axis_name="c", subcore_axis_name="s")


@functools.partial(
    pl.kernel,
    out_type=jax.ShapeDtypeStruct((B, EMBED), jnp.float32),
    mesh=_mesh,
    scratch_types=[
        pltpu.VMEM((NCHUNK, C), jnp.int32),
        pltpu.VMEM((C, EMBED), jnp.float32),
        pltpu.SemaphoreType.DMA,
    ],
)
def _gather_kernel(idx_hbm, table_hbm, out_hbm, idx_v, rows_v, sem):
    wid = lax.axis_index("s") * NC + lax.axis_index("c")
    base = wid * BPW
    pltpu.sync_copy(idx_hbm.at[wid], idx_v)

    def chunk(j, carry):
        pltpu.async_copy(table_hbm.at[idx_v.at[j]], rows_v, sem).wait()
        pltpu.sync_copy(rows_v, out_hbm.at[pl.ds(base + j * C, C)])
        return carry

    lax.fori_loop(0, NCHUNK, chunk, 0)


def kernel(indices, table):
    idx = indices.reshape(NW, NCHUNK, C).astype(jnp.int32)
    out = _gather_kernel(idx, table)
    return out.reshape(BATCH, SEQ, EMBED)


# speedup vs baseline: 3.3460x; 1.1288x over previous
import functools

import jax
import jax.numpy as jnp
from jax import lax
from jax.experimental import pallas as pl
from jax.experimental.pallas import tpu as pltpu
from jax.experimental.pallas import tpu_sc as plsc

VOCAB = 100000
EMBED = 128
BATCH = 4096
SEQ = 50

NC = 2
NS = 16
NW = NC * NS
B = BATCH * SEQ
BPW = B // NW
C = 128
NCHUNK = BPW // C
NBUF = 5
K = 2
NG = NCHUNK // NBUF

_mesh = plsc.VectorSubcoreMesh(core_axis_name="c", subcore_axis_name="s")


@functools.partial(
    pl.kernel,
    out_type=jax.ShapeDtypeStruct((B, EMBED), jnp.float32),
    mesh=_mesh,
    scratch_types=(
        [pltpu.VMEM((NCHUNK, C), jnp.int32)]
        + [pltpu.VMEM((C, EMBED), jnp.float32) for _ in range(NBUF)]
        + [pltpu.SemaphoreType.DMA for _ in range(2 * NBUF)]
    ),
)
def _gather_kernel(idx_hbm, table_hbm, out_hbm, idx_v, *scr):
    rows = scr[:NBUF]
    gsem = scr[NBUF:2 * NBUF]
    osem = scr[2 * NBUF:]
    wid = lax.axis_index("s") * NC + lax.axis_index("c")
    base = wid * BPW

    pltpu.sync_copy(idx_hbm.at[wid], idx_v)

    def start_gather(j, b):
        pltpu.async_copy(table_hbm.at[idx_v.at[j]], rows[b], gsem[b])

    def wait_gather(b):
        pltpu.make_async_copy(table_hbm.at[pl.ds(0, C)], rows[b], gsem[b]).wait()

    def start_out(i, b):
        pltpu.async_copy(rows[b], out_hbm.at[pl.ds(base + i * C, C)], osem[b])

    def wait_out(b):
        pltpu.make_async_copy(rows[b], out_hbm.at[pl.ds(base, C)], osem[b]).wait()

    for j in range(NBUF):
        start_gather(j, j)
        if j >= K:
            i = j - K
            wait_gather(i % NBUF)
            start_out(i, i % NBUF)

    def group(g, carry):
        j0 = g * NBUF
        for b in range(NBUF):
            wait_out(b)
            start_gather(j0 + b, b)
            bo = (b - K) % NBUF
            wait_gather(bo)
            start_out(j0 + b - K, bo)
        return carry

    lax.fori_loop(1, NG, group, 0)

    for t in range(K):
        i = NCHUNK - K + t
        wait_gather(i % NBUF)
        start_out(i, i % NBUF)
    for b in range(NBUF):
        wait_out(b)


def kernel(indices, table):
    idx = indices.reshape(NW, NCHUNK, C).astype(jnp.int32)
    out = _gather_kernel(idx, table)
    return out.reshape(BATCH, SEQ, EMBED)


# speedup vs baseline: 5.9461x; 1.7771x over previous
import functools

import jax
import jax.numpy as jnp
from jax import lax
from jax.experimental import pallas as pl
from jax.experimental.pallas import tpu as pltpu
from jax.experimental.pallas import tpu_sc as plsc

VOCAB = 100000
EMBED = 128
BATCH = 4096
SEQ = 50

NC = 2
NS = 16
NW = NC * NS
BPW = BATCH // NW
NBUF = 8
K = 3
NG = BPW // NBUF

_mesh = plsc.VectorSubcoreMesh(core_axis_name="c", subcore_axis_name="s")


@functools.partial(
    pl.kernel,
    out_type=jax.ShapeDtypeStruct((BATCH, SEQ, EMBED), jnp.float32),
    mesh=_mesh,
    compiler_params=pltpu.CompilerParams(use_tc_tiling_on_sc=True),
    scratch_types=(
        [pltpu.VMEM((BPW, SEQ), jnp.int32)]
        + [pltpu.VMEM((SEQ, EMBED), jnp.float32) for _ in range(NBUF)]
        + [pltpu.SemaphoreType.DMA for _ in range(2 * NBUF)]
    ),
)
def _gather_kernel(idx_hbm, table_hbm, out_hbm, idx_v, *scr):
    rows = scr[:NBUF]
    gsem = scr[NBUF:2 * NBUF]
    osem = scr[2 * NBUF:]
    wid = lax.axis_index("s") * NC + lax.axis_index("c")
    base = wid * BPW

    pltpu.sync_copy(idx_hbm.at[wid], idx_v)

    def start_gather(j, b):
        pltpu.async_copy(table_hbm.at[idx_v.at[j]], rows[b], gsem[b])

    def wait_gather(b):
        pltpu.make_async_copy(out_hbm.at[0], rows[b], gsem[b]).wait()

    def start_out(j, b):
        pltpu.async_copy(rows[b], out_hbm.at[base + j], osem[b])

    def wait_out(b):
        pltpu.make_async_copy(rows[b], out_hbm.at[0], osem[b]).wait()

    for j in range(NBUF):
        start_gather(j, j)
        if j >= K:
            i = j - K
            wait_gather(i % NBUF)
            start_out(i, i % NBUF)

    def group(g, carry):
        j0 = g * NBUF
        for b in range(NBUF):
            wait_out(b)
            start_gather(j0 + b, b)
            bo = (b - K) % NBUF
            wait_gather(bo)
            start_out(j0 + b - K, bo)
        return carry

    lax.fori_loop(1, NG, group, 0)

    for t in range(K):
        i = BPW - K + t
        wait_gather(i % NBUF)
        start_out(i, i % NBUF)
    for b in range(NBUF):
        wait_out(b)


def kernel(indices, table):
    idx = indices.reshape(NW, BPW, SEQ).astype(jnp.int32)
    return _gather_kernel(idx, table)


# speedup vs baseline: 10.6455x; 1.7903x over previous
import functools

import jax
import jax.numpy as jnp
from jax import lax
from jax.experimental import pallas as pl
from jax.experimental.pallas import tpu as pltpu
from jax.experimental.pallas import tpu_sc as plsc

VOCAB = 100000
EMBED = 128
BATCH = 4096
SEQ = 50

NC = 2
NS = 16
NW = NC * NS
C = BATCH // NW
NBUF = 6
K = 3
NG = SEQ // NBUF

_mesh = plsc.VectorSubcoreMesh(core_axis_name="c", subcore_axis_name="s")


@functools.partial(
    pl.kernel,
    out_type=jax.ShapeDtypeStruct((SEQ, BATCH, EMBED), jnp.float32),
    mesh=_mesh,
    scratch_types=(
        [pltpu.VMEM((SEQ, C), jnp.int32)]
        + [pltpu.VMEM((C, EMBED), jnp.float32) for _ in range(NBUF)]
        + [pltpu.SemaphoreType.DMA for _ in range(2 * NBUF)]
    ),
)
def _gather_kernel(idx_hbm, table_hbm, out_hbm, idx_v, *scr):
    rows = scr[:NBUF]
    gsem = scr[NBUF:2 * NBUF]
    osem = scr[2 * NBUF:]
    wid = lax.axis_index("s") * NC + lax.axis_index("c")
    base = wid * C

    pltpu.sync_copy(idx_hbm.at[wid], idx_v)

    def start_gather(j, b):
        pltpu.async_copy(table_hbm.at[idx_v.at[j]], rows[b], gsem[b])

    def wait_gather(b):
        pltpu.make_async_copy(table_hbm.at[pl.ds(0, C)], rows[b], gsem[b]).wait()

    def start_out(j, b):
        pltpu.async_copy(rows[b], out_hbm.at[j, pl.ds(base, C)], osem[b])

    def wait_out(b):
        pltpu.make_async_copy(rows[b], out_hbm.at[0, pl.ds(base, C)], osem[b]).wait()

    def step(j_pat, j_dyn):
        b = j_pat % NBUF
        if j_pat >= NBUF:
            wait_out(b)
        start_gather(j_dyn, b)
        if j_pat >= K:
            bo = (j_pat - K) % NBUF
            wait_gather(bo)
            start_out(j_dyn - K, bo)

    for j in range(NBUF):
        step(j, j)

    def group(g, carry):
        for b in range(NBUF):
            step(NBUF + b, g * NBUF + b)
        return carry

    lax.fori_loop(1, NG, group, 0)

    for j in range(NG * NBUF, SEQ):
        step(NBUF + (j % NBUF), j)

    for t in range(K):
        i = SEQ - K + t
        wait_gather(i % NBUF)
        start_out(i, i % NBUF)
    for b in range(NBUF):
        wait_out(b)


def kernel(indices, table):
    idx = jnp.transpose(
        indices.astype(jnp.int32).reshape(NW, C, SEQ), (0, 2, 1)
    )
    out = _gather_kernel(idx, table)
    return jnp.transpose(out, (1, 0, 2))
